# baseline (speedup 1.0000x reference)
import math

import jax
import jax.numpy as jnp
import numpy as np
from jax.experimental import pallas as pl
from jax.experimental.pallas import tpu as pltpu

B, S, D, H = 1, 2048, 2048, 16
HD = D // H
WIN = 256
HALF = WIN // 2
LN_EPS = 1e-5

RB = 256
QB = 256
NQ = S // QB
KBS = 128
NKB = S // KBS
SPAN = 4 * KBS

_r = np.arange(QB)[:, None]
_c = np.arange(SPAN)[None, :]
_band = np.abs(_r - (_c - KBS)) <= HALF
_pen_int = np.where(_band, 0.0, -1e9).astype(np.float32)
_pen_first = _pen_int.copy()
_pen_first[:, :KBS] = -1e9
_pen_last = _pen_int.copy()
_pen_last[:, 3 * KBS:] = -1e9
_PEN3 = np.stack([_pen_first, _pen_int, _pen_last])


def _proj_cast_kernel(hs_ref, w_ref, scale_ref, b_ref, out_ref, hsbf_ref, w_bf):
    i = pl.program_id(0)

    @pl.when(i == 0)
    def _():
        w_bf[...] = (w_ref[...] * scale_ref[...]).astype(jnp.bfloat16)

    hsb = hs_ref[...].astype(jnp.bfloat16)
    hsbf_ref[...] = hsb
    acc = jnp.dot(hsb, w_bf[...], preferred_element_type=jnp.float32)
    out_ref[...] = (acc + b_ref[...]).astype(jnp.bfloat16)


def _proj_kernel(hs_ref, w_ref, scale_ref, b_ref, out_ref, w_bf):
    i = pl.program_id(0)

    @pl.when(i == 0)
    def _():
        w_bf[...] = (w_ref[...] * scale_ref[...]).astype(jnp.bfloat16)

    acc = jnp.dot(hs_ref[...], w_bf[...], preferred_element_type=jnp.float32)
    out_ref[...] = (acc + b_ref[...]).astype(jnp.bfloat16)


def _attn_out_kernel(q_ref, k0_ref, k1_ref, k2_ref, k3_ref,
                     v0_ref, v1_ref, v2_ref, v3_ref,
                     am0_ref, am1_ref, am2_ref, am3_ref,
                     pen_ref, rowmul_ref, hs_ref, wo_ref, bo_ref,
                     g_ref, bta_ref, out_ref, wo_bf):
    i = pl.program_id(0)

    @pl.when(i == 0)
    def _():
        wo_bf[...] = wo_ref[...].astype(jnp.bfloat16)

    am = jnp.concatenate(
        [am0_ref[...], am1_ref[...], am2_ref[...], am3_ref[...]], axis=1)
    pen = pen_ref[0] + am
    pen_t = pen[:KBS, :3 * KBS]
    pen_b = pen[KBS:, KBS:]
    rowv_t = rowmul_ref[0, :KBS].reshape(KBS, 1)
    rowv_b = rowmul_ref[0, KBS:].reshape(KBS, 1)
    krefs = (k0_ref, k1_ref, k2_ref, k3_ref)
    vrefs = (v0_ref, v1_ref, v2_ref, v3_ref)

    def _half(qh, pen_h, rowv_h, kset, vset):
        s = jnp.concatenate(
            [jax.lax.dot_general(qh, kr[:, kset[0]], (((1,), (1,)), ((), ())),
                                 preferred_element_type=jnp.float32)
             for kr in kset[1]], axis=1)
        e = jnp.exp(jnp.minimum(s + pen_h, 60.0))
        l = jnp.sum(e, axis=-1, keepdims=True)
        eb = e.astype(jnp.bfloat16)
        acc = jnp.dot(eb[:, :KBS], vset[0][:, kset[0]],
                      preferred_element_type=jnp.float32)
        for j in (1, 2):
            acc = acc + jnp.dot(eb[:, j * KBS:(j + 1) * KBS],
                                vset[j][:, kset[0]],
                                preferred_element_type=jnp.float32)
        return acc * (rowv_h / l)

    ctx_parts = []
    for h in range(H):
        sl = slice(h * HD, (h + 1) * HD)
        ctx_t = _half(q_ref[:KBS, sl], pen_t, rowv_t,
                      (sl, krefs[:3]), vrefs[:3])
        ctx_b = _half(q_ref[KBS:, sl], pen_b, rowv_b,
                      (sl, krefs[1:]), vrefs[1:])
        ctx_parts.append(
            jnp.concatenate([ctx_t, ctx_b], axis=0).astype(jnp.bfloat16))

    ctx = jnp.concatenate(ctx_parts, axis=1)
    o = jnp.dot(ctx, wo_bf[...], preferred_element_type=jnp.float32)
    y = o + bo_ref[...] + hs_ref[...]
    mu = jnp.mean(y, axis=-1, keepdims=True)
    yc = y - mu
    var = jnp.mean(yc * yc, axis=-1, keepdims=True)
    y = yc * jax.lax.rsqrt(var + LN_EPS)
    out_ref[...] = y * g_ref[...] + bta_ref[...]


def _proj_cast(hs, w, scale, b):
    return pl.pallas_call(
        _proj_cast_kernel,
        grid=(S // RB,),
        in_specs=[
            pl.BlockSpec((RB, D), lambda i: (i, 0)),
            pl.BlockSpec((D, D), lambda i: (0, 0)),
            pl.BlockSpec((1, D), lambda i: (0, 0)),
            pl.BlockSpec((1, D), lambda i: (0, 0)),
        ],
        out_specs=[
            pl.BlockSpec((RB, D), lambda i: (i, 0)),
            pl.BlockSpec((RB, D), lambda i: (i, 0)),
        ],
        out_shape=[
            jax.ShapeDtypeStruct((S, D), jnp.bfloat16),
            jax.ShapeDtypeStruct((S, D), jnp.bfloat16),
        ],
        scratch_shapes=[pltpu.VMEM((D, D), jnp.bfloat16)],
    )(hs, w, scale.reshape(1, D), b.reshape(1, D))


def _proj(hs_bf, w, scale, b):
    return pl.pallas_call(
        _proj_kernel,
        grid=(S // RB,),
        in_specs=[
            pl.BlockSpec((RB, D), lambda i: (i, 0)),
            pl.BlockSpec((D, D), lambda i: (0, 0)),
            pl.BlockSpec((1, D), lambda i: (0, 0)),
            pl.BlockSpec((1, D), lambda i: (0, 0)),
        ],
        out_specs=pl.BlockSpec((RB, D), lambda i: (i, 0)),
        out_shape=jax.ShapeDtypeStruct((S, D), jnp.bfloat16),
        scratch_shapes=[pltpu.VMEM((D, D), jnp.bfloat16)],
    )(hs_bf, w, scale.reshape(1, D), b.reshape(1, D))


def kernel(hidden_states, attention_mask, layer_head_mask, Wq, bq, Wk, bk, Wv, bv,
           Wo, bo, ln_gamma, ln_beta, is_index_masked, is_index_global_attn,
           is_global_attn):
    hs = hidden_states.reshape(S, D)
    inv = 1.0 / math.sqrt(HD)
    ones = jnp.ones((D,), jnp.float32)
    hm_cols = jnp.repeat(layer_head_mask, HD)

    q, hs_bf = _proj_cast(hs, Wq, ones * inv, bq * inv)
    k = _proj(hs_bf, Wk, ones, bk)
    v = _proj(hs_bf, Wv, hm_cols, bv * hm_cols)

    am = attention_mask.reshape(1, S)
    rowmul = (1.0 - is_index_masked.astype(jnp.float32)).reshape(1, S)
    pen3 = jnp.asarray(_PEN3)

    k0 = pl.BlockSpec((KBS, D), lambda i: (jnp.maximum(2 * i - 1, 0), 0))
    k1 = pl.BlockSpec((KBS, D), lambda i: (2 * i, 0))
    k2 = pl.BlockSpec((KBS, D), lambda i: (2 * i + 1, 0))
    k3 = pl.BlockSpec((KBS, D), lambda i: (jnp.minimum(2 * i + 2, NKB - 1), 0))
    a0 = pl.BlockSpec((1, KBS), lambda i: (0, jnp.maximum(2 * i - 1, 0)))
    a1 = pl.BlockSpec((1, KBS), lambda i: (0, 2 * i))
    a2 = pl.BlockSpec((1, KBS), lambda i: (0, 2 * i + 1))
    a3 = pl.BlockSpec((1, KBS), lambda i: (0, jnp.minimum(2 * i + 2, NKB - 1)))
    pen_spec = pl.BlockSpec(
        (1, QB, SPAN),
        lambda i: (jnp.where(i == 0, 0, jnp.where(i == NQ - 1, 2, 1)), 0, 0))

    y = pl.pallas_call(
        _attn_out_kernel,
        grid=(NQ,),
        in_specs=[
            pl.BlockSpec((QB, D), lambda i: (i, 0)),
            k0, k1, k2, k3, k0, k1, k2, k3,
            a0, a1, a2, a3,
            pen_spec,
            pl.BlockSpec((1, QB), lambda i: (0, i)),
            pl.BlockSpec((QB, D), lambda i: (i, 0)),
            pl.BlockSpec((D, D), lambda i: (0, 0)),
            pl.BlockSpec((1, D), lambda i: (0, 0)),
            pl.BlockSpec((1, D), lambda i: (0, 0)),
            pl.BlockSpec((1, D), lambda i: (0, 0)),
        ],
        out_specs=pl.BlockSpec((QB, D), lambda i: (i, 0)),
        out_shape=jax.ShapeDtypeStruct((S, D), jnp.float32),
        scratch_shapes=[pltpu.VMEM((D, D), jnp.bfloat16)],
    )(q, k, k, k, k, v, v, v, v, am, am, am, am, pen3, rowmul, hs,
      Wo, bo.reshape(1, D), ln_gamma.reshape(1, D), ln_beta.reshape(1, D))

    return y.reshape(B, S, D)


# speedup vs baseline: 1.2162x; 1.2162x over previous
import math

import jax
import jax.numpy as jnp
import numpy as np
from jax.experimental import pallas as pl
from jax.experimental.pallas import tpu as pltpu

B, S, D, H = 1, 2048, 2048, 16
HD = D // H
WIN = 256
HALF = WIN // 2
LN_EPS = 1e-5

RB = 256
QB = 256
NQ = S // QB
KBS = 128
NKB = S // KBS
SPAN = 4 * KBS

_r = np.arange(QB)[:, None]
_c = np.arange(SPAN)[None, :]
_band = np.abs(_r - (_c - KBS)) <= HALF
_pen_int = np.where(_band, 0.0, -1e9).astype(np.float32)
_pen_first = _pen_int.copy()
_pen_first[:, :KBS] = -1e9
_pen_last = _pen_int.copy()
_pen_last[:, 3 * KBS:] = -1e9
_PEN3 = np.stack([_pen_first, _pen_int, _pen_last])


def _proj_cast_kernel(hs_ref, w_ref, scale_ref, b_ref, out_ref, hsbf_ref, w_bf):
    i = pl.program_id(0)

    @pl.when(i == 0)
    def _():
        w_bf[...] = (w_ref[...] * scale_ref[...]).astype(jnp.bfloat16)

    hsb = hs_ref[...].astype(jnp.bfloat16)
    hsbf_ref[...] = hsb
    acc = jnp.dot(hsb, w_bf[...], preferred_element_type=jnp.float32)
    out_ref[...] = (acc + b_ref[...]).astype(jnp.bfloat16)


def _kv_kernel(hs_ref, wk_ref, wv_ref, vscale_ref, bk_ref, bv_ref,
               k_out, v_out, wk_bf, wv_bf):
    i = pl.program_id(0)

    @pl.when(i == 0)
    def _():
        wk_bf[...] = wk_ref[...].astype(jnp.bfloat16)
        wv_bf[...] = (wv_ref[...] * vscale_ref[...]).astype(jnp.bfloat16)

    hsb = hs_ref[...]
    k_out[...] = (jnp.dot(hsb, wk_bf[...], preferred_element_type=jnp.float32)
                  + bk_ref[...]).astype(jnp.bfloat16)
    v_out[...] = (jnp.dot(hsb, wv_bf[...], preferred_element_type=jnp.float32)
                  + bv_ref[...]).astype(jnp.bfloat16)


def _attn_out_kernel(q_ref, k0_ref, k1_ref, k2_ref, k3_ref,
                     v0_ref, v1_ref, v2_ref, v3_ref,
                     am0_ref, am1_ref, am2_ref, am3_ref,
                     pen_ref, rowmul_ref, hs_ref, wo_ref, bo_ref,
                     g_ref, bta_ref, out_ref, wo_bf):
    i = pl.program_id(0)

    @pl.when(i == 0)
    def _():
        wo_bf[...] = wo_ref[...].astype(jnp.bfloat16)

    am = jnp.concatenate(
        [am0_ref[...], am1_ref[...], am2_ref[...], am3_ref[...]], axis=1)
    pen = pen_ref[0] + am
    rowv = rowmul_ref[0, :].reshape(QB, 1)
    krefs = (k0_ref, k1_ref, k2_ref, k3_ref)
    vrefs = (v0_ref, v1_ref, v2_ref, v3_ref)

    ctx_parts = []
    for h in range(H):
        sl = slice(h * HD, (h + 1) * HD)
        qh = q_ref[:, sl]
        s = jnp.concatenate(
            [jax.lax.dot_general(qh, kr[:, sl], (((1,), (1,)), ((), ())),
                                 preferred_element_type=jnp.float32)
             for kr in krefs], axis=1)
        e = jnp.exp(jnp.minimum(s + pen, 60.0))
        l = jnp.sum(e, axis=-1, keepdims=True)
        eb = e.astype(jnp.bfloat16)
        acc = jnp.dot(eb[:, :KBS], vrefs[0][:, sl],
                      preferred_element_type=jnp.float32)
        for j in range(1, 4):
            acc = acc + jnp.dot(eb[:, j * KBS:(j + 1) * KBS], vrefs[j][:, sl],
                                preferred_element_type=jnp.float32)
        ctx_parts.append((acc * (rowv / l)).astype(jnp.bfloat16))

    ctx = jnp.concatenate(ctx_parts, axis=1)
    o = jnp.dot(ctx, wo_bf[...], preferred_element_type=jnp.float32)
    y = o + bo_ref[...] + hs_ref[...]
    mu = jnp.mean(y, axis=-1, keepdims=True)
    yc = y - mu
    var = jnp.mean(yc * yc, axis=-1, keepdims=True)
    y = yc * jax.lax.rsqrt(var + LN_EPS)
    out_ref[...] = y * g_ref[...] + bta_ref[...]


def _proj_cast(hs, w, scale, b):
    return pl.pallas_call(
        _proj_cast_kernel,
        grid=(S // RB,),
        in_specs=[
            pl.BlockSpec((RB, D), lambda i: (i, 0)),
            pl.BlockSpec((D, D), lambda i: (0, 0)),
            pl.BlockSpec((1, D), lambda i: (0, 0)),
            pl.BlockSpec((1, D), lambda i: (0, 0)),
        ],
        out_specs=[
            pl.BlockSpec((RB, D), lambda i: (i, 0)),
            pl.BlockSpec((RB, D), lambda i: (i, 0)),
        ],
        out_shape=[
            jax.ShapeDtypeStruct((S, D), jnp.bfloat16),
            jax.ShapeDtypeStruct((S, D), jnp.bfloat16),
        ],
        scratch_shapes=[pltpu.VMEM((D, D), jnp.bfloat16)],
    )(hs, w, scale.reshape(1, D), b.reshape(1, D))


def _kv_proj(hs_bf, wk, wv, vscale, bk, bv):
    return pl.pallas_call(
        _kv_kernel,
        grid=(S // RB,),
        in_specs=[
            pl.BlockSpec((RB, D), lambda i: (i, 0)),
            pl.BlockSpec((D, D), lambda i: (0, 0)),
            pl.BlockSpec((D, D), lambda i: (0, 0)),
            pl.BlockSpec((1, D), lambda i: (0, 0)),
            pl.BlockSpec((1, D), lambda i: (0, 0)),
            pl.BlockSpec((1, D), lambda i: (0, 0)),
        ],
        out_specs=[
            pl.BlockSpec((RB, D), lambda i: (i, 0)),
            pl.BlockSpec((RB, D), lambda i: (i, 0)),
        ],
        out_shape=[
            jax.ShapeDtypeStruct((S, D), jnp.bfloat16),
            jax.ShapeDtypeStruct((S, D), jnp.bfloat16),
        ],
        scratch_shapes=[pltpu.VMEM((D, D), jnp.bfloat16),
                        pltpu.VMEM((D, D), jnp.bfloat16)],
    )(hs_bf, wk, wv, vscale.reshape(1, D), bk.reshape(1, D), bv.reshape(1, D))


def kernel(hidden_states, attention_mask, layer_head_mask, Wq, bq, Wk, bk, Wv, bv,
           Wo, bo, ln_gamma, ln_beta, is_index_masked, is_index_global_attn,
           is_global_attn):
    hs = hidden_states.reshape(S, D)
    inv = 1.0 / math.sqrt(HD)
    ones = jnp.ones((D,), jnp.float32)
    hm_cols = jnp.repeat(layer_head_mask, HD)

    q, hs_bf = _proj_cast(hs, Wq, ones * inv, bq * inv)
    k, v = _kv_proj(hs_bf, Wk, Wv, hm_cols, bk, bv * hm_cols)

    am = attention_mask.reshape(1, S)
    rowmul = (1.0 - is_index_masked.astype(jnp.float32)).reshape(1, S)
    pen3 = jnp.asarray(_PEN3)

    k0 = pl.BlockSpec((KBS, D), lambda i: (jnp.maximum(2 * i - 1, 0), 0))
    k1 = pl.BlockSpec((KBS, D), lambda i: (2 * i, 0))
    k2 = pl.BlockSpec((KBS, D), lambda i: (2 * i + 1, 0))
    k3 = pl.BlockSpec((KBS, D), lambda i: (jnp.minimum(2 * i + 2, NKB - 1), 0))
    a0 = pl.BlockSpec((1, KBS), lambda i: (0, jnp.maximum(2 * i - 1, 0)))
    a1 = pl.BlockSpec((1, KBS), lambda i: (0, 2 * i))
    a2 = pl.BlockSpec((1, KBS), lambda i: (0, 2 * i + 1))
    a3 = pl.BlockSpec((1, KBS), lambda i: (0, jnp.minimum(2 * i + 2, NKB - 1)))
    pen_spec = pl.BlockSpec(
        (1, QB, SPAN),
        lambda i: (jnp.where(i == 0, 0, jnp.where(i == NQ - 1, 2, 1)), 0, 0))

    y = pl.pallas_call(
        _attn_out_kernel,
        grid=(NQ,),
        in_specs=[
            pl.BlockSpec((QB, D), lambda i: (i, 0)),
            k0, k1, k2, k3, k0, k1, k2, k3,
            a0, a1, a2, a3,
            pen_spec,
            pl.BlockSpec((1, QB), lambda i: (0, i)),
            pl.BlockSpec((QB, D), lambda i: (i, 0)),
            pl.BlockSpec((D, D), lambda i: (0, 0)),
            pl.BlockSpec((1, D), lambda i: (0, 0)),
            pl.BlockSpec((1, D), lambda i: (0, 0)),
            pl.BlockSpec((1, D), lambda i: (0, 0)),
        ],
        out_specs=pl.BlockSpec((QB, D), lambda i: (i, 0)),
        out_shape=jax.ShapeDtypeStruct((S, D), jnp.float32),
        scratch_shapes=[pltpu.VMEM((D, D), jnp.bfloat16)],
    )(q, k, k, k, k, v, v, v, v, am, am, am, am, pen3, rowmul, hs,
      Wo, bo.reshape(1, D), ln_gamma.reshape(1, D), ln_beta.reshape(1, D))

    return y.reshape(B, S, D)


# speedup vs baseline: 1.2280x; 1.0097x over previous
import math

import jax
import jax.numpy as jnp
import numpy as np
from jax.experimental import pallas as pl
from jax.experimental.pallas import tpu as pltpu

B, S, D, H = 1, 2048, 2048, 16
HD = D // H
WIN = 256
HALF = WIN // 2
LN_EPS = 1e-5

RB = 256
QB = 256
NQ = S // QB
KBS = 128
NKB = S // KBS
SPAN = 4 * KBS

_r = np.arange(QB)[:, None]
_c = np.arange(SPAN)[None, :]
_band = np.abs(_r - (_c - KBS)) <= HALF
_pen_int = np.where(_band, 0.0, -1e9).astype(np.float32)
_pen_first = _pen_int.copy()
_pen_first[:, :KBS] = -1e9
_pen_last = _pen_int.copy()
_pen_last[:, 3 * KBS:] = -1e9
_PEN3 = np.stack([_pen_first, _pen_int, _pen_last])


def _proj_cast_kernel(hs_ref, w_ref, scale_ref, b_ref, out_ref, hsbf_ref, w_bf):
    i = pl.program_id(0)

    @pl.when(i == 0)
    def _():
        w_bf[...] = (w_ref[...] * scale_ref[...]).astype(jnp.bfloat16)

    hsb = hs_ref[...].astype(jnp.bfloat16)
    hsbf_ref[...] = hsb
    acc = jnp.dot(hsb, w_bf[...], preferred_element_type=jnp.float32)
    out_ref[...] = (acc + b_ref[...]).astype(jnp.bfloat16)


def _kv_kernel(hs_ref, wk_ref, wv_ref, vscale_ref, bk_ref, bv_ref,
               k_out, v_out, wk_bf, wv_bf):
    i = pl.program_id(0)

    @pl.when(i == 0)
    def _():
        wk_bf[...] = wk_ref[...].astype(jnp.bfloat16)
        wv_bf[...] = (wv_ref[...] * vscale_ref[...]).astype(jnp.bfloat16)

    hsb = hs_ref[...]
    k_out[...] = (jnp.dot(hsb, wk_bf[...], preferred_element_type=jnp.float32)
                  + bk_ref[...]).astype(jnp.bfloat16)
    v_out[...] = (jnp.dot(hsb, wv_bf[...], preferred_element_type=jnp.float32)
                  + bv_ref[...]).astype(jnp.bfloat16)


def _attn_out_kernel(q_ref, k0_ref, k1_ref, k2_ref, k3_ref,
                     v0_ref, v1_ref, v2_ref, v3_ref,
                     am0_ref, am1_ref, am2_ref, am3_ref,
                     pen_ref, rowmul_ref, hs_ref, wo_ref, bo_ref,
                     g_ref, bta_ref, out_ref, wo_bf):
    i = pl.program_id(0)

    @pl.when(i == 0)
    def _():
        wo_bf[...] = wo_ref[...].astype(jnp.bfloat16)

    am = jnp.concatenate(
        [am0_ref[...], am1_ref[...], am2_ref[...], am3_ref[...]], axis=1)
    pen = pen_ref[0] + am
    rowv = rowmul_ref[0, :].reshape(QB, 1)
    krefs = (k0_ref, k1_ref, k2_ref, k3_ref)
    vrefs = (v0_ref, v1_ref, v2_ref, v3_ref)

    ctx_parts = []
    for h in range(H):
        sl = slice(h * HD, (h + 1) * HD)
        qh = q_ref[:, sl]
        s = jnp.concatenate(
            [jax.lax.dot_general(qh, kr[:, sl], (((1,), (1,)), ((), ())),
                                 preferred_element_type=jnp.float32)
             for kr in krefs], axis=1)
        e = jnp.exp(jnp.minimum(s + pen, 60.0))
        l = jnp.sum(e, axis=-1, keepdims=True)
        eb = e.astype(jnp.bfloat16)
        acc = jnp.dot(eb[:, :KBS], vrefs[0][:, sl],
                      preferred_element_type=jnp.float32)
        for j in range(1, 4):
            acc = acc + jnp.dot(eb[:, j * KBS:(j + 1) * KBS], vrefs[j][:, sl],
                                preferred_element_type=jnp.float32)
        ctx_parts.append((acc * (rowv / l)).astype(jnp.bfloat16))

    ctx = jnp.concatenate(ctx_parts, axis=1)
    o = jnp.dot(ctx, wo_bf[...], preferred_element_type=jnp.float32)
    y = o + bo_ref[...] + hs_ref[...].astype(jnp.float32)
    mu = jnp.mean(y, axis=-1, keepdims=True)
    yc = y - mu
    var = jnp.mean(yc * yc, axis=-1, keepdims=True)
    y = yc * jax.lax.rsqrt(var + LN_EPS)
    out_ref[...] = y * g_ref[...] + bta_ref[...]


def _proj_cast(hs, w, scale, b):
    return pl.pallas_call(
        _proj_cast_kernel,
        grid=(S // RB,),
        in_specs=[
            pl.BlockSpec((RB, D), lambda i: (i, 0)),
            pl.BlockSpec((D, D), lambda i: (0, 0)),
            pl.BlockSpec((1, D), lambda i: (0, 0)),
            pl.BlockSpec((1, D), lambda i: (0, 0)),
        ],
        out_specs=[
            pl.BlockSpec((RB, D), lambda i: (i, 0)),
            pl.BlockSpec((RB, D), lambda i: (i, 0)),
        ],
        out_shape=[
            jax.ShapeDtypeStruct((S, D), jnp.bfloat16),
            jax.ShapeDtypeStruct((S, D), jnp.bfloat16),
        ],
        scratch_shapes=[pltpu.VMEM((D, D), jnp.bfloat16)],
    )(hs, w, scale.reshape(1, D), b.reshape(1, D))


def _kv_proj(hs_bf, wk, wv, vscale, bk, bv):
    return pl.pallas_call(
        _kv_kernel,
        grid=(S // RB,),
        in_specs=[
            pl.BlockSpec((RB, D), lambda i: (i, 0)),
            pl.BlockSpec((D, D), lambda i: (0, 0)),
            pl.BlockSpec((D, D), lambda i: (0, 0)),
            pl.BlockSpec((1, D), lambda i: (0, 0)),
            pl.BlockSpec((1, D), lambda i: (0, 0)),
            pl.BlockSpec((1, D), lambda i: (0, 0)),
        ],
        out_specs=[
            pl.BlockSpec((RB, D), lambda i: (i, 0)),
            pl.BlockSpec((RB, D), lambda i: (i, 0)),
        ],
        out_shape=[
            jax.ShapeDtypeStruct((S, D), jnp.bfloat16),
            jax.ShapeDtypeStruct((S, D), jnp.bfloat16),
        ],
        scratch_shapes=[pltpu.VMEM((D, D), jnp.bfloat16),
                        pltpu.VMEM((D, D), jnp.bfloat16)],
    )(hs_bf, wk, wv, vscale.reshape(1, D), bk.reshape(1, D), bv.reshape(1, D))


def kernel(hidden_states, attention_mask, layer_head_mask, Wq, bq, Wk, bk, Wv, bv,
           Wo, bo, ln_gamma, ln_beta, is_index_masked, is_index_global_attn,
           is_global_attn):
    hs = hidden_states.reshape(S, D)
    inv = 1.0 / math.sqrt(HD)
    ones = jnp.ones((D,), jnp.float32)
    hm_cols = jnp.repeat(layer_head_mask, HD)

    q, hs_bf = _proj_cast(hs, Wq, ones * inv, bq * inv)
    k, v = _kv_proj(hs_bf, Wk, Wv, hm_cols, bk, bv * hm_cols)

    am = attention_mask.reshape(1, S)
    rowmul = (1.0 - is_index_masked.astype(jnp.float32)).reshape(1, S)
    pen3 = jnp.asarray(_PEN3)

    k0 = pl.BlockSpec((KBS, D), lambda i: (jnp.maximum(2 * i - 1, 0), 0))
    k1 = pl.BlockSpec((KBS, D), lambda i: (2 * i, 0))
    k2 = pl.BlockSpec((KBS, D), lambda i: (2 * i + 1, 0))
    k3 = pl.BlockSpec((KBS, D), lambda i: (jnp.minimum(2 * i + 2, NKB - 1), 0))
    a0 = pl.BlockSpec((1, KBS), lambda i: (0, jnp.maximum(2 * i - 1, 0)))
    a1 = pl.BlockSpec((1, KBS), lambda i: (0, 2 * i))
    a2 = pl.BlockSpec((1, KBS), lambda i: (0, 2 * i + 1))
    a3 = pl.BlockSpec((1, KBS), lambda i: (0, jnp.minimum(2 * i + 2, NKB - 1)))
    pen_spec = pl.BlockSpec(
        (1, QB, SPAN),
        lambda i: (jnp.where(i == 0, 0, jnp.where(i == NQ - 1, 2, 1)), 0, 0))

    y = pl.pallas_call(
        _attn_out_kernel,
        grid=(NQ,),
        in_specs=[
            pl.BlockSpec((QB, D), lambda i: (i, 0)),
            k0, k1, k2, k3, k0, k1, k2, k3,
            a0, a1, a2, a3,
            pen_spec,
            pl.BlockSpec((1, QB), lambda i: (0, i)),
            pl.BlockSpec((QB, D), lambda i: (i, 0)),
            pl.BlockSpec((D, D), lambda i: (0, 0)),
            pl.BlockSpec((1, D), lambda i: (0, 0)),
            pl.BlockSpec((1, D), lambda i: (0, 0)),
            pl.BlockSpec((1, D), lambda i: (0, 0)),
        ],
        out_specs=pl.BlockSpec((QB, D), lambda i: (i, 0)),
        out_shape=jax.ShapeDtypeStruct((S, D), jnp.float32),
        scratch_shapes=[pltpu.VMEM((D, D), jnp.bfloat16)],
    )(q, k, k, k, k, v, v, v, v, am, am, am, am, pen3, rowmul, hs_bf,
      Wo, bo.reshape(1, D), ln_gamma.reshape(1, D), ln_beta.reshape(1, D))

    return y.reshape(B, S, D)
